# baseline (speedup 1.0000x reference)
import functools
import math

import jax
import jax.numpy as jnp
from jax import lax
from jax.experimental import pallas as pl
from jax.experimental.pallas import tpu as pltpu
from jax.experimental.pallas import tpu_sc as plsc

D_MODEL = 64
LANES = 16
NUM_CORES = 2
NUM_SUBCORES = 16
NUM_WORKERS = NUM_CORES * NUM_SUBCORES
GROUP = 128


def _build(num_groups):
    groups_per_worker = num_groups // NUM_WORKERS
    mesh = plsc.VectorSubcoreMesh(
        core_axis_name="c", subcore_axis_name="s",
        num_cores=NUM_CORES, num_subcores=NUM_SUBCORES)

    @functools.partial(
        pl.kernel,
        out_type=jax.ShapeDtypeStruct((num_groups, GROUP, D_MODEL), jnp.float32),
        mesh=mesh,
        scratch_types=[
            pltpu.VMEM((groups_per_worker, GROUP), jnp.int32),
            pltpu.VMEM((1, GROUP, D_MODEL), jnp.float32),
            pltpu.SemaphoreType.DMA,
        ],
        compiler_params=pltpu.CompilerParams(use_tc_tiling_on_sc=False),
    )
    def emb_kernel(table_hbm, idx_hbm, out_hbm, idx_all, rows, gsem):
        wid = lax.axis_index("s") * NUM_CORES + lax.axis_index("c")
        g0 = wid * groups_per_worker
        pltpu.sync_copy(idx_hbm.at[pl.ds(g0, groups_per_worker)], idx_all)

        @pl.loop(0, groups_per_worker)
        def _chunk(c):
            pltpu.async_copy(table_hbm.at[idx_all.at[c]], rows.at[0], gsem).wait()

            @pl.loop(0, GROUP)
            def _row(r):
                for dd in range(D_MODEL // LANES):
                    sl = pl.ds(dd * LANES, LANES)
                    rows[0, r, sl] = rows[0, r, sl] * 8.0

            pltpu.sync_copy(rows, out_hbm.at[pl.ds(g0 + c, 1)])

    return emb_kernel


def kernel(x, table):
    s0, s1 = x.shape
    total = s0 * s1
    num_groups = total // GROUP
    idx = x.reshape(total).astype(jnp.int32).reshape(num_groups, GROUP)
    emb = _build(num_groups)(table, idx)
    return emb.reshape(s0, s1, D_MODEL)


# speedup vs baseline: 1.2024x; 1.2024x over previous
import functools
import math

import jax
import jax.numpy as jnp
from jax import lax
from jax.experimental import pallas as pl
from jax.experimental.pallas import tpu as pltpu
from jax.experimental.pallas import tpu_sc as plsc

D_MODEL = 64
LANES = 16
NUM_CORES = 2
NUM_SUBCORES = 16
NUM_WORKERS = NUM_CORES * NUM_SUBCORES
GROUP = 128


NBUF = 4
PDIST = 2


def _build(num_groups):
    groups_per_worker = num_groups // NUM_WORKERS
    mesh = plsc.VectorSubcoreMesh(
        core_axis_name="c", subcore_axis_name="s",
        num_cores=NUM_CORES, num_subcores=NUM_SUBCORES)

    @functools.partial(
        pl.kernel,
        out_type=jax.ShapeDtypeStruct((num_groups, GROUP, D_MODEL), jnp.float32),
        mesh=mesh,
        scratch_types=[
            pltpu.VMEM((groups_per_worker, GROUP), jnp.int32),
            pltpu.VMEM((NBUF, GROUP, D_MODEL), jnp.float32),
            [pltpu.SemaphoreType.DMA] * NBUF,
            [pltpu.SemaphoreType.DMA] * NBUF,
        ],
        compiler_params=pltpu.CompilerParams(use_tc_tiling_on_sc=False),
    )
    def emb_kernel(table_hbm, idx_hbm, out_hbm, idx_all, rows, gsem, osem):
        wid = lax.axis_index("s") * NUM_CORES + lax.axis_index("c")
        g0 = wid * groups_per_worker
        n = groups_per_worker
        pltpu.sync_copy(idx_hbm.at[pl.ds(g0, n)], idx_all)

        def gather_start(c, s):
            pltpu.async_copy(table_hbm.at[idx_all.at[c]], rows.at[s], gsem[s])

        def gather_wait(c, s):
            pltpu.make_async_copy(
                table_hbm.at[idx_all.at[c]], rows.at[s], gsem[s]).wait()

        def store_start(c, s):
            pltpu.async_copy(
                rows.at[pl.ds(s, 1)], out_hbm.at[pl.ds(g0 + c, 1)], osem[s])

        def store_wait(c, s):
            pltpu.make_async_copy(
                rows.at[pl.ds(s, 1)], out_hbm.at[pl.ds(g0 + c, 1)], osem[s]).wait()

        for c in range(PDIST):
            gather_start(c, c % NBUF)

        @pl.loop(0, n, step=NBUF)
        def _chunks(c0):
            for b in range(NBUF):
                c = c0 + b
                s = b
                sp = (b + PDIST) % NBUF
                cp = c + PDIST

                @pl.when(jnp.logical_and(cp < n, cp >= NBUF))
                def _():
                    store_wait(cp - NBUF, sp)

                @pl.when(cp < n)
                def _():
                    gather_start(cp, sp)

                gather_wait(c, s)

                @pl.loop(0, GROUP, unroll=4)
                def _row(r):
                    for dd in range(D_MODEL // LANES):
                        sl = pl.ds(dd * LANES, LANES)
                        rows[s, r, sl] = rows[s, r, sl] * 8.0

                store_start(c, s)

        for b in range(NBUF):
            store_wait(n - NBUF + b, b)

    return emb_kernel


def kernel(x, table):
    s0, s1 = x.shape
    total = s0 * s1
    num_groups = total // GROUP
    idx = x.reshape(total).astype(jnp.int32).reshape(num_groups, GROUP)
    emb = _build(num_groups)(table, idx)
    return emb.reshape(s0, s1, D_MODEL)
